# baseline (speedup 1.0000x reference)
import functools

import jax
import jax.numpy as jnp
from jax import lax
from jax.experimental import pallas as pl
from jax.experimental.pallas import tpu as pltpu
from jax.experimental.pallas import tpu_sc as plsc

RBF_MIN = 0.0
RBF_MAX = 30.0
LOG2 = 0.6931471805599453

NC = 2
NS = 16
NW = NC * NS
SUB = 128


def _ssp(x):
    return jnp.maximum(x, 0.0) + jnp.log1p(jnp.exp(-jnp.abs(x))) - LOG2


def _edge_filters(d, fnet_W1, fnet_b1):
    E = d.shape[0]
    L, NRBF, F = fnet_W1.shape
    BLK = 512
    step = (RBF_MAX - RBF_MIN) / (NRBF - 1)
    coeff = -0.5 / step**2

    def body(d_ref, w1_ref, b1_ref, out_ref):
        dcol = d_ref[...]
        offs = (lax.broadcasted_iota(jnp.int32, (1, NRBF), 1).astype(jnp.float32)
                * step + RBF_MIN)
        f = jnp.exp(coeff * (dcol - offs) ** 2)
        for l in range(L):
            out_ref[l] = _ssp(
                jnp.dot(f, w1_ref[l], preferred_element_type=jnp.float32)
                + b1_ref[l][None, :])

    return pl.pallas_call(
        body,
        grid=(E // BLK,),
        in_specs=[
            pl.BlockSpec((BLK, 1), lambda i: (i, 0)),
            pl.BlockSpec((L, NRBF, F), lambda i: (0, 0, 0)),
            pl.BlockSpec((L, F), lambda i: (0, 0)),
        ],
        out_specs=pl.BlockSpec((L, BLK, F), lambda i: (0, i, 0)),
        out_shape=jax.ShapeDtypeStruct((L, E, F), jnp.float32),
    )(d.reshape(E, 1), fnet_W1, fnet_b1)


def _sc_segment_sum(wf, idx1, zeros, N):
    L, E, F = wf.shape
    NSUB = E // SUB
    T0 = NSUB // NW
    REM = NSUB - T0 * NW
    RLO = (N // NS) // 8 * 8
    NHI = (N - RLO * NS) // 8

    mesh = plsc.VectorSubcoreMesh(core_axis_name="c", subcore_axis_name="s")

    @functools.partial(
        pl.kernel,
        out_type=(jax.ShapeDtypeStruct((L, NC, N, F), jnp.float32),
                  jax.ShapeDtypeStruct((NC, N, F), jnp.float32)),
        mesh=mesh,
        scratch_types=[
            pltpu.VMEM((SUB,), jnp.int32),
            pltpu.VMEM((SUB,), jnp.int32),
            pltpu.VMEM((SUB, F), jnp.float32),
            pltpu.VMEM((SUB, F), jnp.float32),
            pltpu.VMEM((SUB, F), jnp.float32),
            pltpu.VMEM_SHARED((N, F), jnp.float32),
            pltpu.SemaphoreType.DMA,
            pltpu.SemaphoreType.DMA,
        ],
    )
    def seg(wf_hbm, idx_hbm, z_hbm, out_hbm, cnt_hbm,
            idxb0, idxb1, rows0, rows1, ones_v, acc, sem0, sem1):
        c = lax.axis_index("c")
        s = lax.axis_index("s")
        w = s * NC + c
        tw = T0 + jnp.where(w < REM, 1, 0)
        idx_bufs = (idxb0, idxb1)
        row_bufs = (rows0, rows1)
        sems = (sem0, sem1)
        row0 = jnp.where(s < NHI, s * (RLO + 8), NHI * 8 + s * RLO)

        one16 = jnp.full((16,), 1.0, jnp.float32)

        def _fill(i, carry):
            for j in range(F // 16):
                ones_v[i, pl.ds(j * 16, 16)] = one16
            return carry

        lax.fori_loop(0, SUB, _fill, 0)

        for p in range(L + 1):
            is_cnt = p == L

            @pl.when(s < NHI)
            def _():
                pltpu.sync_copy(z_hbm, acc.at[pl.ds(row0, RLO + 8)])

            @pl.when(s >= NHI)
            def _():
                pltpu.sync_copy(z_hbm.at[pl.ds(0, RLO)], acc.at[pl.ds(row0, RLO)])

            plsc.subcore_barrier()

            def start(b, t):
                r = w + NW * t
                pltpu.async_copy(idx_hbm.at[pl.ds(r * SUB, SUB)], idx_bufs[b],
                                 sems[b])
                if not is_cnt:
                    pltpu.async_copy(wf_hbm.at[p].at[pl.ds(r * SUB, SUB)],
                                     row_bufs[b], sems[b])

            def drain(b):
                pltpu.make_async_copy(idx_hbm.at[pl.ds(0, SUB)], idx_bufs[b],
                                      sems[b]).wait()
                if not is_cnt:
                    pltpu.make_async_copy(wf_hbm.at[0].at[pl.ds(0, SUB)],
                                          row_bufs[b], sems[b]).wait()

            def scat(b):
                src = ones_v if is_cnt else row_bufs[b]
                pltpu.sync_copy(src, acc.at[idx_bufs[b]], add=True)

            start(0, 0)
            start(1, 1)

            def gbody(g, carry):
                for b in range(2):
                    t = 2 * g + b
                    drain(b)
                    scat(b)
                    nxt = t + 2

                    @pl.when(nxt < tw)
                    def _():
                        start(b, nxt)
                return carry

            lax.fori_loop(0, T0 // 2, gbody, 0)

            @pl.when(w < REM)
            def _():
                drain(0)
                scat(0)

            plsc.subcore_barrier()

            dst = cnt_hbm.at[c] if is_cnt else out_hbm.at[p].at[c]

            @pl.when(s < NHI)
            def _():
                pltpu.sync_copy(acc.at[pl.ds(row0, RLO + 8)],
                                dst.at[pl.ds(row0, RLO + 8)])

            @pl.when(s >= NHI)
            def _():
                pltpu.sync_copy(acc.at[pl.ds(row0, RLO)],
                                dst.at[pl.ds(row0, RLO)])

            plsc.subcore_barrier()

    return seg(wf, idx1, zeros)


def _node_net(Z2, emb_p, in2f_W, parts, cnts, fnet_W2, fnet_b2,
              f2out_W1, f2out_b1, f2out_W2, f2out_b2, out_W1, out_W2, out_b2):
    L, NCp, N, F = parts.shape
    MZ = emb_p.shape[0]
    H = out_W1.shape[1]
    BLK = 1000

    def body(z_ref, emb_ref, in2f_ref, parts_ref, cnt_ref, fw2_ref, fb2_ref,
             w1_ref, b1_ref, w2_ref, b2_ref, ow1_ref, ow2_ref, ob2_ref,
             out_ref):
        z = z_ref[...]
        ids = lax.broadcasted_iota(jnp.int32, (1, MZ), 1)
        oh = (z == ids).astype(jnp.float32)
        x = jnp.dot(oh, emb_ref[...], preferred_element_type=jnp.float32)
        cnt = (cnt_ref[0] + cnt_ref[1])[:, 0:1]
        for l in range(L):
            h = jnp.dot(x, in2f_ref[l], preferred_element_type=jnp.float32)
            su = parts_ref[l, 0] + parts_ref[l, 1]
            s = (jnp.dot(su, fw2_ref[l], preferred_element_type=jnp.float32)
                 + cnt * fb2_ref[l][None, :])
            agg = h * s
            t = _ssp(jnp.dot(agg, w1_ref[l], preferred_element_type=jnp.float32)
                     + b1_ref[l][None, :])
            x = x + jnp.dot(t, w2_ref[l], preferred_element_type=jnp.float32) \
                + b2_ref[l][None, :]
        y = _ssp(jnp.dot(x, ow1_ref[...], preferred_element_type=jnp.float32))
        out_ref[...] = (jnp.dot(y, ow2_ref[...], preferred_element_type=jnp.float32)
                        + ob2_ref[0, 0])

    return pl.pallas_call(
        body,
        grid=(Z2.shape[0] // BLK,),
        in_specs=[
            pl.BlockSpec((BLK, 1), lambda i: (i, 0)),
            pl.BlockSpec((MZ, F), lambda i: (0, 0)),
            pl.BlockSpec((L, F, F), lambda i: (0, 0, 0)),
            pl.BlockSpec((L, NCp, BLK, F), lambda i: (0, 0, i, 0)),
            pl.BlockSpec((NCp, BLK, F), lambda i: (0, i, 0)),
            pl.BlockSpec((L, F, F), lambda i: (0, 0, 0)),
            pl.BlockSpec((L, F), lambda i: (0, 0)),
            pl.BlockSpec((L, F, F), lambda i: (0, 0, 0)),
            pl.BlockSpec((L, F), lambda i: (0, 0)),
            pl.BlockSpec((L, F, F), lambda i: (0, 0, 0)),
            pl.BlockSpec((L, F), lambda i: (0, 0)),
            pl.BlockSpec((F, H), lambda i: (0, 0)),
            pl.BlockSpec((H, 1), lambda i: (0, 0)),
            pl.BlockSpec((1, 1), lambda i: (0, 0)),
        ],
        out_specs=pl.BlockSpec((BLK, 1), lambda i: (i, 0)),
        out_shape=jax.ShapeDtypeStruct((N, 1), jnp.float32),
    )(Z2, emb_p, in2f_W, parts, cnts, fnet_W2, fnet_b2, f2out_W1, f2out_b1,
      f2out_W2, f2out_b2, out_W1, out_W2, out_b2.reshape(1, 1))


def kernel(Z, d, idx_j, emb, in2f_W, fnet_W1, fnet_b1, fnet_W2, fnet_b2,
           f2out_W1, f2out_b1, f2out_W2, f2out_b2, out_W1, out_W2, out_b2):
    N = Z.shape[0]
    E = d.shape[0]
    F = emb.shape[1]

    wf = _edge_filters(d, fnet_W1, fnet_b1)

    idx1 = idx_j.astype(jnp.int32)
    zrows = (N // NS) // 8 * 8 + 8
    zeros = jnp.zeros((zrows, F), jnp.float32)
    parts, cnts = _sc_segment_sum(wf, idx1, zeros, N)

    emb_p = jnp.zeros((128, F), jnp.float32).at[:emb.shape[0]].set(emb)
    out = _node_net(Z.astype(jnp.int32).reshape(N, 1), emb_p, in2f_W, parts,
                    cnts, fnet_W2, fnet_b2, f2out_W1, f2out_b1, f2out_W2,
                    f2out_b2, out_W1, out_W2, out_b2)
    return out.reshape(N)


# speedup vs baseline: 1.1410x; 1.1410x over previous
import functools

import jax
import jax.numpy as jnp
from jax import lax
from jax.experimental import pallas as pl
from jax.experimental.pallas import tpu as pltpu
from jax.experimental.pallas import tpu_sc as plsc

RBF_MIN = 0.0
RBF_MAX = 30.0
LOG2 = 0.6931471805599453

NC = 2
NS = 16
NW = NC * NS
SUB = 128
NCHUNK = 2


def _ssp(x):
    return jnp.maximum(x, 0.0) + jnp.log1p(jnp.exp(-jnp.abs(x))) - LOG2


def _edge_filters(d, fnet_W1, fnet_b1):
    EC = d.shape[0]
    L, NRBF, F = fnet_W1.shape
    BLK = 640
    step = (RBF_MAX - RBF_MIN) / (NRBF - 1)
    coeff = -0.5 / step**2

    def body(d_ref, w1_ref, b1_ref, out_ref):
        dcol = d_ref[...]
        offs = (lax.broadcasted_iota(jnp.int32, (1, NRBF), 1).astype(jnp.float32)
                * step + RBF_MIN)
        f = jnp.exp(coeff * (dcol - offs) ** 2)
        for l in range(L):
            out_ref[l] = _ssp(
                jnp.dot(f, w1_ref[l], preferred_element_type=jnp.float32)
                + b1_ref[l][None, :])

    return pl.pallas_call(
        body,
        grid=(EC // BLK,),
        in_specs=[
            pl.BlockSpec((BLK, 1), lambda i: (i, 0)),
            pl.BlockSpec((L, NRBF, F), lambda i: (0, 0, 0)),
            pl.BlockSpec((L, F), lambda i: (0, 0)),
        ],
        out_specs=pl.BlockSpec((L, BLK, F), lambda i: (0, i, 0)),
        out_shape=jax.ShapeDtypeStruct((L, EC, F), jnp.float32),
    )(d.reshape(EC, 1), fnet_W1, fnet_b1)


def _subcore_rows(s, N):
    RLO = (N // NS) // 8 * 8
    NHI = (N - RLO * NS) // 8
    row0 = jnp.where(s < NHI, s * (RLO + 8), NHI * 8 + s * RLO)
    return row0, RLO, NHI


def _zero_acc(s, z_hbm, acc, N):
    row0, RLO, NHI = _subcore_rows(s, N)

    @pl.when(s < NHI)
    def _():
        pltpu.sync_copy(z_hbm, acc.at[pl.ds(row0, RLO + 8)])

    @pl.when(s >= NHI)
    def _():
        pltpu.sync_copy(z_hbm.at[pl.ds(0, RLO)], acc.at[pl.ds(row0, RLO)])


def _dump_acc(s, acc, dst, N):
    row0, RLO, NHI = _subcore_rows(s, N)

    @pl.when(s < NHI)
    def _():
        pltpu.sync_copy(acc.at[pl.ds(row0, RLO + 8)],
                        dst.at[pl.ds(row0, RLO + 8)])

    @pl.when(s >= NHI)
    def _():
        pltpu.sync_copy(acc.at[pl.ds(row0, RLO)], dst.at[pl.ds(row0, RLO)])


def _sc_segment_sum(wf, idx1, zeros, N):
    L, EC, F = wf.shape
    NSUB = EC // SUB
    T0 = NSUB // NW
    REM = NSUB - T0 * NW
    GMAX = (T0 + (1 if REM else 0) + 1) // 2

    mesh = plsc.VectorSubcoreMesh(core_axis_name="c", subcore_axis_name="s")

    @functools.partial(
        pl.kernel,
        out_type=jax.ShapeDtypeStruct((L, NC, N, F), jnp.float32),
        mesh=mesh,
        scratch_types=[
            pltpu.VMEM((SUB,), jnp.int32),
            pltpu.VMEM((SUB,), jnp.int32),
            pltpu.VMEM((SUB, F), jnp.float32),
            pltpu.VMEM((SUB, F), jnp.float32),
            pltpu.VMEM_SHARED((N, F), jnp.float32),
            pltpu.SemaphoreType.DMA,
            pltpu.SemaphoreType.DMA,
        ],
    )
    def seg(wf_hbm, idx_hbm, z_hbm, out_hbm,
            idxb0, idxb1, rows0, rows1, acc, sem0, sem1):
        c = lax.axis_index("c")
        s = lax.axis_index("s")
        w = s * NC + c
        tw = T0 + jnp.where(w < REM, 1, 0)
        idx_bufs = (idxb0, idxb1)
        row_bufs = (rows0, rows1)
        sems = (sem0, sem1)

        for p in range(L):
            _zero_acc(s, z_hbm, acc, N)
            plsc.subcore_barrier()

            def start(b, t):
                r = w + NW * t
                pltpu.async_copy(idx_hbm.at[pl.ds(r * SUB, SUB)], idx_bufs[b],
                                 sems[b])
                pltpu.async_copy(wf_hbm.at[p].at[pl.ds(r * SUB, SUB)],
                                 row_bufs[b], sems[b])

            def drain(b):
                pltpu.make_async_copy(idx_hbm.at[pl.ds(0, SUB)], idx_bufs[b],
                                      sems[b]).wait()
                pltpu.make_async_copy(wf_hbm.at[0].at[pl.ds(0, SUB)],
                                      row_bufs[b], sems[b]).wait()

            start(0, 0)

            @pl.when(1 < tw)
            def _():
                start(1, 1)

            def gbody(g, carry):
                for b in range(2):
                    t = 2 * g + b

                    @pl.when(t < tw)
                    def _():
                        drain(b)
                        pltpu.sync_copy(row_bufs[b], acc.at[idx_bufs[b]],
                                        add=True)

                        @pl.when(t + 2 < tw)
                        def _():
                            start(b, t + 2)
                return carry

            lax.fori_loop(0, GMAX, gbody, 0)
            plsc.subcore_barrier()
            _dump_acc(s, acc, out_hbm.at[p].at[c], N)
            plsc.subcore_barrier()

    return seg(wf, idx1, zeros)


def _sc_counts(idx1, zeros, N, F):
    E = idx1.shape[0]
    NSUB = E // SUB
    T0 = NSUB // NW
    REM = NSUB - T0 * NW
    GMAX = (T0 + (1 if REM else 0) + 1) // 2

    mesh = plsc.VectorSubcoreMesh(core_axis_name="c", subcore_axis_name="s")

    @functools.partial(
        pl.kernel,
        out_type=jax.ShapeDtypeStruct((NC, N, F), jnp.float32),
        mesh=mesh,
        scratch_types=[
            pltpu.VMEM((SUB,), jnp.int32),
            pltpu.VMEM((SUB,), jnp.int32),
            pltpu.VMEM((SUB, F), jnp.float32),
            pltpu.VMEM_SHARED((N, F), jnp.float32),
            pltpu.SemaphoreType.DMA,
            pltpu.SemaphoreType.DMA,
        ],
    )
    def cntk(idx_hbm, z_hbm, out_hbm, idxb0, idxb1, ones_v, acc, sem0, sem1):
        c = lax.axis_index("c")
        s = lax.axis_index("s")
        w = s * NC + c
        tw = T0 + jnp.where(w < REM, 1, 0)
        idx_bufs = (idxb0, idxb1)
        sems = (sem0, sem1)

        one16 = jnp.full((16,), 1.0, jnp.float32)

        def _fill(i, carry):
            for j in range(F // 16):
                ones_v[i, pl.ds(j * 16, 16)] = one16
            return carry

        lax.fori_loop(0, SUB, _fill, 0)

        _zero_acc(s, z_hbm, acc, N)
        plsc.subcore_barrier()

        def start(b, t):
            r = w + NW * t
            pltpu.async_copy(idx_hbm.at[pl.ds(r * SUB, SUB)], idx_bufs[b],
                             sems[b])

        def drain(b):
            pltpu.make_async_copy(idx_hbm.at[pl.ds(0, SUB)], idx_bufs[b],
                                  sems[b]).wait()

        start(0, 0)

        @pl.when(1 < tw)
        def _():
            start(1, 1)

        def gbody(g, carry):
            for b in range(2):
                t = 2 * g + b

                @pl.when(t < tw)
                def _():
                    drain(b)
                    pltpu.sync_copy(ones_v, acc.at[idx_bufs[b]], add=True)

                    @pl.when(t + 2 < tw)
                    def _():
                        start(b, t + 2)
            return carry

        lax.fori_loop(0, GMAX, gbody, 0)
        plsc.subcore_barrier()
        _dump_acc(s, acc, out_hbm.at[c], N)
        plsc.subcore_barrier()

    return cntk(idx1, zeros)


def _node_net(Z2, emb_p, in2f_W, parts_list, cnts, fnet_W2, fnet_b2,
              f2out_W1, f2out_b1, f2out_W2, f2out_b2, out_W1, out_W2, out_b2):
    K = len(parts_list)
    L, NCp, N, F = parts_list[0].shape
    MZ = emb_p.shape[0]
    H = out_W1.shape[1]
    BLK = 1000

    def body(*refs):
        z_ref, emb_ref, in2f_ref = refs[0], refs[1], refs[2]
        parts_refs = refs[3:3 + K]
        (cnt_ref, fw2_ref, fb2_ref, w1_ref, b1_ref, w2_ref, b2_ref,
         ow1_ref, ow2_ref, ob2_ref, out_ref) = refs[3 + K:]
        z = z_ref[...]
        ids = lax.broadcasted_iota(jnp.int32, (1, MZ), 1)
        oh = (z == ids).astype(jnp.float32)
        x = jnp.dot(oh, emb_ref[...], preferred_element_type=jnp.float32)
        cnt = (cnt_ref[0] + cnt_ref[1])[:, 0:1]
        for l in range(L):
            h = jnp.dot(x, in2f_ref[l], preferred_element_type=jnp.float32)
            su = parts_refs[0][l, 0] + parts_refs[0][l, 1]
            for k in range(1, K):
                su = su + parts_refs[k][l, 0] + parts_refs[k][l, 1]
            s = (jnp.dot(su, fw2_ref[l], preferred_element_type=jnp.float32)
                 + cnt * fb2_ref[l][None, :])
            agg = h * s
            t = _ssp(jnp.dot(agg, w1_ref[l], preferred_element_type=jnp.float32)
                     + b1_ref[l][None, :])
            x = x + jnp.dot(t, w2_ref[l], preferred_element_type=jnp.float32) \
                + b2_ref[l][None, :]
        y = _ssp(jnp.dot(x, ow1_ref[...], preferred_element_type=jnp.float32))
        out_ref[...] = (jnp.dot(y, ow2_ref[...], preferred_element_type=jnp.float32)
                        + ob2_ref[0, 0])

    return pl.pallas_call(
        body,
        grid=(Z2.shape[0] // BLK,),
        in_specs=[
            pl.BlockSpec((BLK, 1), lambda i: (i, 0)),
            pl.BlockSpec((MZ, F), lambda i: (0, 0)),
            pl.BlockSpec((L, F, F), lambda i: (0, 0, 0)),
        ] + [
            pl.BlockSpec((L, NCp, BLK, F), lambda i: (0, 0, i, 0))
            for _ in range(K)
        ] + [
            pl.BlockSpec((NCp, BLK, F), lambda i: (0, i, 0)),
            pl.BlockSpec((L, F, F), lambda i: (0, 0, 0)),
            pl.BlockSpec((L, F), lambda i: (0, 0)),
            pl.BlockSpec((L, F, F), lambda i: (0, 0, 0)),
            pl.BlockSpec((L, F), lambda i: (0, 0)),
            pl.BlockSpec((L, F, F), lambda i: (0, 0, 0)),
            pl.BlockSpec((L, F), lambda i: (0, 0)),
            pl.BlockSpec((F, H), lambda i: (0, 0)),
            pl.BlockSpec((H, 1), lambda i: (0, 0)),
            pl.BlockSpec((1, 1), lambda i: (0, 0)),
        ],
        out_specs=pl.BlockSpec((BLK, 1), lambda i: (i, 0)),
        out_shape=jax.ShapeDtypeStruct((Z2.shape[0], 1), jnp.float32),
    )(Z2, emb_p, in2f_W, *parts_list, cnts, fnet_W2, fnet_b2, f2out_W1,
      f2out_b1, f2out_W2, f2out_b2, out_W1, out_W2, out_b2.reshape(1, 1))


def kernel(Z, d, idx_j, emb, in2f_W, fnet_W1, fnet_b1, fnet_W2, fnet_b2,
           f2out_W1, f2out_b1, f2out_W2, f2out_b2, out_W1, out_W2, out_b2):
    N = Z.shape[0]
    E = d.shape[0]
    F = emb.shape[1]

    idx1 = idx_j.astype(jnp.int32)
    zrows = (N // NS) // 8 * 8 + 8
    zeros = jnp.zeros((zrows, F), jnp.float32)

    cnts = _sc_counts(idx1, zeros, N, F)

    EC = E // NCHUNK
    parts_list = []
    for k in range(NCHUNK):
        wf_k = _edge_filters(d[k * EC:(k + 1) * EC], fnet_W1, fnet_b1)
        parts_list.append(
            _sc_segment_sum(wf_k, idx1[k * EC:(k + 1) * EC], zeros, N))

    emb_p = jnp.zeros((128, F), jnp.float32).at[:emb.shape[0]].set(emb)
    out = _node_net(Z.astype(jnp.int32).reshape(N, 1), emb_p, in2f_W,
                    parts_list, cnts, fnet_W2, fnet_b2, f2out_W1, f2out_b1,
                    f2out_W2, f2out_b2, out_W1, out_W2, out_b2)
    return out.reshape(N)


# speedup vs baseline: 1.5430x; 1.3523x over previous
import functools

import jax
import jax.numpy as jnp
from jax import lax
from jax.experimental import pallas as pl
from jax.experimental.pallas import tpu as pltpu
from jax.experimental.pallas import tpu_sc as plsc

RBF_MIN = 0.0
RBF_MAX = 30.0
LOG2 = 0.6931471805599453

NC = 2
NS = 16
NW = NC * NS
SUB = 128
NCHUNK = 2


def _ssp(x):
    return jnp.maximum(x, 0.0) + jnp.log1p(jnp.exp(-jnp.abs(x))) - LOG2


def _edge_filters(d, fnet_W1, fnet_b1):
    EC = d.shape[0]
    L, NRBF, F = fnet_W1.shape
    JB = 5
    BLK = JB * 128
    step = (RBF_MAX - RBF_MIN) / (NRBF - 1)
    coeff = -0.5 / step**2

    def body(d_ref, w1_ref, b1_ref, out_ref):
        offs = (lax.broadcasted_iota(jnp.int32, (NRBF, 1), 0).astype(jnp.float32)
                * step + RBF_MIN)
        for j in range(JB):
            drow = d_ref[0, j:j + 1, :]
            ft = jnp.exp(coeff * (drow - offs) ** 2)
            for l in range(L):
                u = lax.dot_general(ft, w1_ref[l], (((0,), (0,)), ((), ())),
                                    preferred_element_type=jnp.float32)
                out_ref[l, pl.ds(j * 128, 128), :] = _ssp(u + b1_ref[l][None, :])

    return pl.pallas_call(
        body,
        grid=(EC // BLK,),
        in_specs=[
            pl.BlockSpec((1, JB, 128), lambda i: (i, 0, 0)),
            pl.BlockSpec((L, NRBF, F), lambda i: (0, 0, 0)),
            pl.BlockSpec((L, F), lambda i: (0, 0)),
        ],
        out_specs=pl.BlockSpec((L, BLK, F), lambda i: (0, i, 0)),
        out_shape=jax.ShapeDtypeStruct((L, EC, F), jnp.float32),
    )(d.reshape(EC // BLK, JB, 128), fnet_W1, fnet_b1)


def _subcore_rows(s, N):
    RLO = (N // NS) // 8 * 8
    NHI = (N - RLO * NS) // 8
    row0 = jnp.where(s < NHI, s * (RLO + 8), NHI * 8 + s * RLO)
    return row0, RLO, NHI


def _zero_acc(s, z_hbm, acc, N):
    row0, RLO, NHI = _subcore_rows(s, N)

    @pl.when(s < NHI)
    def _():
        pltpu.sync_copy(z_hbm, acc.at[pl.ds(row0, RLO + 8)])

    @pl.when(s >= NHI)
    def _():
        pltpu.sync_copy(z_hbm.at[pl.ds(0, RLO)], acc.at[pl.ds(row0, RLO)])


def _dump_acc(s, acc, dst, N):
    row0, RLO, NHI = _subcore_rows(s, N)

    @pl.when(s < NHI)
    def _():
        pltpu.sync_copy(acc.at[pl.ds(row0, RLO + 8)],
                        dst.at[pl.ds(row0, RLO + 8)])

    @pl.when(s >= NHI)
    def _():
        pltpu.sync_copy(acc.at[pl.ds(row0, RLO)], dst.at[pl.ds(row0, RLO)])


def _sc_segment_sum(wf, idx1, idx_full, zeros, N, with_counts):
    L, EC, F = wf.shape
    mesh = plsc.VectorSubcoreMesh(core_axis_name="c", subcore_axis_name="s")

    out_type = jax.ShapeDtypeStruct((L, NC, N, F), jnp.float32)
    if with_counts:
        out_type = (out_type, jax.ShapeDtypeStruct((NC, N, F), jnp.float32))

    def rounds(nsub):
        t0 = nsub // NW
        rem = nsub - t0 * NW
        gmax = (t0 + (1 if rem else 0) + 1) // 2
        return t0, rem, gmax

    @functools.partial(
        pl.kernel,
        out_type=out_type,
        mesh=mesh,
        scratch_types=[
            pltpu.VMEM((SUB,), jnp.int32),
            pltpu.VMEM((SUB,), jnp.int32),
            pltpu.VMEM((SUB, F), jnp.float32),
            pltpu.VMEM((SUB, F), jnp.float32),
            pltpu.VMEM_SHARED((N, F), jnp.float32),
            pltpu.SemaphoreType.DMA,
            pltpu.SemaphoreType.DMA,
        ],
    )
    def seg(*args):
        if with_counts:
            (wf_hbm, idx_hbm, idxf_hbm, z_hbm, out_hbm, cnt_hbm,
             idxb0, idxb1, rows0, rows1, acc, sem0, sem1) = args
        else:
            (wf_hbm, idx_hbm, z_hbm, out_hbm,
             idxb0, idxb1, rows0, rows1, acc, sem0, sem1) = args
        c = lax.axis_index("c")
        s = lax.axis_index("s")
        w = s * NC + c
        idx_bufs = (idxb0, idxb1)
        row_bufs = (rows0, rows1)
        sems = (sem0, sem1)

        npass = L + (1 if with_counts else 0)
        for p in range(npass):
            is_cnt = p == L
            src_idx = idxf_hbm if is_cnt else idx_hbm
            T0, REM, GMAX = rounds(
                (idx_full if is_cnt else idx1).shape[0] // SUB)
            tw = T0 + jnp.where(w < REM, 1, 0)

            if is_cnt:
                one16 = jnp.full((16,), 1.0, jnp.float32)

                def _fill(i, carry):
                    for j in range(F // 16):
                        rows0[i, pl.ds(j * 16, 16)] = one16
                    return carry

                lax.fori_loop(0, SUB, _fill, 0)

            _zero_acc(s, z_hbm, acc, N)
            plsc.subcore_barrier()

            def start(b, t):
                r = w + NW * t
                pltpu.async_copy(src_idx.at[pl.ds(r * SUB, SUB)], idx_bufs[b],
                                 sems[b])
                if not is_cnt:
                    pltpu.async_copy(wf_hbm.at[p].at[pl.ds(r * SUB, SUB)],
                                     row_bufs[b], sems[b])

            def drain(b):
                pltpu.make_async_copy(src_idx.at[pl.ds(0, SUB)], idx_bufs[b],
                                      sems[b]).wait()
                if not is_cnt:
                    pltpu.make_async_copy(wf_hbm.at[0].at[pl.ds(0, SUB)],
                                          row_bufs[b], sems[b]).wait()

            def scat(b):
                src = rows0 if is_cnt else row_bufs[b]
                pltpu.sync_copy(src, acc.at[idx_bufs[b]], add=True)

            start(0, 0)

            @pl.when(1 < tw)
            def _():
                start(1, 1)

            def gbody(g, carry):
                for b in range(2):
                    t = 2 * g + b

                    @pl.when(t < tw)
                    def _():
                        drain(b)
                        scat(b)

                        @pl.when(t + 2 < tw)
                        def _():
                            start(b, t + 2)
                return carry

            lax.fori_loop(0, GMAX, gbody, 0)
            plsc.subcore_barrier()
            dst = cnt_hbm.at[c] if is_cnt else out_hbm.at[p].at[c]
            _dump_acc(s, acc, dst, N)
            plsc.subcore_barrier()

    if with_counts:
        return seg(wf, idx1, idx_full, zeros)
    return seg(wf, idx1, zeros)


def _node_net(Z2, emb_p, in2f_W, parts_list, cnts, fnet_W2, fnet_b2,
              f2out_W1, f2out_b1, f2out_W2, f2out_b2, out_W1, out_W2, out_b2):
    K = len(parts_list)
    L, NCp, N, F = parts_list[0].shape
    MZ = emb_p.shape[0]
    H = out_W1.shape[1]
    BLK = 1000

    def body(*refs):
        z_ref, emb_ref, in2f_ref = refs[0], refs[1], refs[2]
        parts_refs = refs[3:3 + K]
        (cnt_ref, fw2_ref, fb2_ref, w1_ref, b1_ref, w2_ref, b2_ref,
         ow1_ref, ow2_ref, ob2_ref, out_ref) = refs[3 + K:]
        z = z_ref[...]
        ids = lax.broadcasted_iota(jnp.int32, (1, MZ), 1)
        oh = (z == ids).astype(jnp.float32)
        x = jnp.dot(oh, emb_ref[...], preferred_element_type=jnp.float32)
        cnt = (cnt_ref[0] + cnt_ref[1])[:, 0:1]
        for l in range(L):
            h = jnp.dot(x, in2f_ref[l], preferred_element_type=jnp.float32)
            su = parts_refs[0][l, 0] + parts_refs[0][l, 1]
            for k in range(1, K):
                su = su + parts_refs[k][l, 0] + parts_refs[k][l, 1]
            s = (jnp.dot(su, fw2_ref[l], preferred_element_type=jnp.float32)
                 + cnt * fb2_ref[l][None, :])
            agg = h * s
            t = _ssp(jnp.dot(agg, w1_ref[l], preferred_element_type=jnp.float32)
                     + b1_ref[l][None, :])
            x = x + jnp.dot(t, w2_ref[l], preferred_element_type=jnp.float32) \
                + b2_ref[l][None, :]
        y = _ssp(jnp.dot(x, ow1_ref[...], preferred_element_type=jnp.float32))
        out_ref[...] = (jnp.dot(y, ow2_ref[...], preferred_element_type=jnp.float32)
                        + ob2_ref[0, 0])

    return pl.pallas_call(
        body,
        grid=(Z2.shape[0] // BLK,),
        in_specs=[
            pl.BlockSpec((BLK, 1), lambda i: (i, 0)),
            pl.BlockSpec((MZ, F), lambda i: (0, 0)),
            pl.BlockSpec((L, F, F), lambda i: (0, 0, 0)),
        ] + [
            pl.BlockSpec((L, NCp, BLK, F), lambda i: (0, 0, i, 0))
            for _ in range(K)
        ] + [
            pl.BlockSpec((NCp, BLK, F), lambda i: (0, i, 0)),
            pl.BlockSpec((L, F, F), lambda i: (0, 0, 0)),
            pl.BlockSpec((L, F), lambda i: (0, 0)),
            pl.BlockSpec((L, F, F), lambda i: (0, 0, 0)),
            pl.BlockSpec((L, F), lambda i: (0, 0)),
            pl.BlockSpec((L, F, F), lambda i: (0, 0, 0)),
            pl.BlockSpec((L, F), lambda i: (0, 0)),
            pl.BlockSpec((F, H), lambda i: (0, 0)),
            pl.BlockSpec((H, 1), lambda i: (0, 0)),
            pl.BlockSpec((1, 1), lambda i: (0, 0)),
        ],
        out_specs=pl.BlockSpec((BLK, 1), lambda i: (i, 0)),
        out_shape=jax.ShapeDtypeStruct((Z2.shape[0], 1), jnp.float32),
    )(Z2, emb_p, in2f_W, *parts_list, cnts, fnet_W2, fnet_b2, f2out_W1,
      f2out_b1, f2out_W2, f2out_b2, out_W1, out_W2, out_b2.reshape(1, 1))


def kernel(Z, d, idx_j, emb, in2f_W, fnet_W1, fnet_b1, fnet_W2, fnet_b2,
           f2out_W1, f2out_b1, f2out_W2, f2out_b2, out_W1, out_W2, out_b2):
    N = Z.shape[0]
    E = d.shape[0]
    F = emb.shape[1]

    idx1 = idx_j.astype(jnp.int32)
    zrows = (N // NS) // 8 * 8 + 8
    zeros = jnp.zeros((zrows, F), jnp.float32)

    EC = E // NCHUNK
    parts_list = []
    cnts = None
    for k in range(NCHUNK):
        wf_k = _edge_filters(d[k * EC:(k + 1) * EC], fnet_W1, fnet_b1)
        res = _sc_segment_sum(wf_k, idx1[k * EC:(k + 1) * EC], idx1, zeros,
                              N, with_counts=(k == 0))
        if k == 0:
            parts_k, cnts = res
        else:
            parts_k = res
        parts_list.append(parts_k)

    emb_p = jnp.zeros((128, F), jnp.float32).at[:emb.shape[0]].set(emb)
    out = _node_net(Z.astype(jnp.int32).reshape(N, 1), emb_p, in2f_W,
                    parts_list, cnts, fnet_W2, fnet_b2, f2out_W1, f2out_b1,
                    f2out_W2, f2out_b2, out_W1, out_W2, out_b2)
    return out.reshape(N)
